# baseline (speedup 1.0000x reference)
import functools

import jax
import jax.numpy as jnp
from jax import lax
from jax.experimental import pallas as pl
from jax.experimental.pallas import tpu as pltpu
from jax.experimental.pallas import tpu_sc as plsc

DM = 1024
NS = 65536
KT = 32
L = 2048
NC1 = NS // 1024
NCH = NS // 128
NW = 32
RPW = L // NW
WCAP = 96
CAPA = 128
NEG = -3.0e38


def _a_body(q_ref, k_ref, s_ref, cm_ref, c1_ref):
    s = lax.dot_general(q_ref[...], k_ref[...], (((1,), (1,)), ((), ())),
                        preferred_element_type=jnp.float32)
    s_ref[0] = s
    cms = []
    for j in range(8):
        blk = s[:, j * 128:(j + 1) * 128]
        cms.append(jnp.max(blk, axis=1, keepdims=True))
    cm = jnp.concatenate(cms, axis=1)
    cm_ref[0] = cm
    c1_ref[0] = jnp.max(cm, axis=1, keepdims=True)


def _stage_a(qb, kb):
    return pl.pallas_call(
        _a_body,
        grid=(NC1,),
        in_specs=[pl.BlockSpec((L, DM), lambda i: (0, 0)),
                  pl.BlockSpec((1024, DM), lambda i: (i, 0))],
        out_specs=[pl.BlockSpec((1, L, 1024), lambda i: (i, 0, 0)),
                   pl.BlockSpec((1, L, 8), lambda i: (i, 0, 0)),
                   pl.BlockSpec((1, L, 1), lambda i: (i, 0, 0))],
        out_shape=[jax.ShapeDtypeStruct((NC1, L, 1024), jnp.float32),
                   jax.ShapeDtypeStruct((NC1, L, 8), jnp.float32),
                   jax.ShapeDtypeStruct((NC1, L, 1), jnp.float32)],
    )(qb, kb)


def _a2_body(c1k_ref, t0_ref):
    m = c1k_ref[...]
    li = lax.broadcasted_iota(jnp.int32, (L, NC1), 1)
    cur = None
    for _ in range(KT):
        cur = jnp.max(m, axis=1, keepdims=True)
        sel = jnp.where(m == cur, li, NC1)
        first = jnp.min(sel, axis=1, keepdims=True)
        m = jnp.where(li == first, NEG, m)
    t0_ref[...] = jnp.broadcast_to(cur, (L, 16))


def _stage_a2(c1k):
    return pl.pallas_call(
        _a2_body,
        out_shape=jax.ShapeDtypeStruct((L, 16), jnp.float32),
    )(c1k)


def _scal(v):
    return jnp.max(v, axis=0) if v.ndim else v


def _bc_body(sc_ref, cm_hbm, c1k_hbm, t0_hbm, vals_hbm, out_hbm,
             cm_v, c1k_v, t0_v, cand_v, candg_v, gath_v, wl_v, av_v, ap_v,
             selv_v, selw_v, sels_v, vrows_v, acc_v, sem1, sem2):
    wid = lax.axis_index("s") * 2 + lax.axis_index("c")
    iota = lax.broadcasted_iota(jnp.int32, (16,), 0)
    iota8 = jnp.bitwise_and(iota, 7)

    def do_row(r, _carry):
        row = wid * RPW + r
        pltpu.sync_copy(cm_hbm.at[row], cm_v)
        pltpu.sync_copy(c1k_hbm.at[row], c1k_v)
        pltpu.sync_copy(t0_hbm.at[row], t0_v)
        t0 = t0_v[...]

        def scan_body(j, off):
            v = c1k_v[pl.ds(j * 16, 16)]
            m = v >= t0
            ids = j * 16 + iota
            idx = jnp.minimum(off + jnp.cumsum(m.astype(jnp.int32)) - 1,
                              KT - 1)
            plsc.store_scatter(cand_v, [idx], ids, mask=m)
            cnt = _scal(plsc.all_reduce_population_count(m))
            return off + cnt
        lax.fori_loop(0, NC1 // 16, scan_body, 0)
        for j in range(KT // 16):
            candg_v[pl.ds(j * 16, 16)] = (
                cand_v[pl.ds(j * 16, 16)] * L + row)

        pltpu.async_copy(sc_ref.at[candg_v], gath_v, sem1).wait()

        def wl_body(j, off):
            c = plsc.load_gather(cand_v, [jnp.full((16,), j, jnp.int32)])
            vals16 = plsc.load_gather(cm_v, [c * 8 + iota8])
            m = (vals16 >= t0) & (iota < 8)
            wpos = j * 8 + iota8
            idx = jnp.minimum(off + jnp.cumsum(m.astype(jnp.int32)) - 1,
                              WCAP - 1)
            plsc.store_scatter(wl_v, [idx], wpos, mask=m)
            cnt = _scal(plsc.all_reduce_population_count(m))
            return off + cnt
        nw = lax.fori_loop(0, KT, wl_body, 0)
        nw = jnp.minimum(nw, WCAP)

        for j in range(CAPA // 16):
            av_v[pl.ds(j * 16, 16)] = jnp.full((16,), NEG, jnp.float32)

        def slot_body(tv, off):
            t = tv // 8
            v8 = tv % 8
            w = plsc.load_gather(wl_v, [jnp.full((16,), t, jnp.int32)])
            j = w >> 3
            k = jnp.bitwise_and(w, 7)
            col = k * 128 + v8 * 16 + iota
            v = plsc.load_gather(gath_v, [j, col])
            m = v >= t0
            c1 = plsc.load_gather(cand_v, [j])
            slots = c1 * 1024 + col
            idx = jnp.minimum(off + jnp.cumsum(m.astype(jnp.int32)) - 1,
                              CAPA - 1)
            plsc.store_scatter(av_v, [idx], v, mask=m)
            plsc.store_scatter(ap_v, [idx], slots, mask=m)
            cnt = _scal(plsc.all_reduce_population_count(m))
            return off + cnt
        lax.fori_loop(0, nw * 8, slot_body, 0)

        def peel_body(i, _c):
            vs = [av_v[pl.ds(j * 16, 16)] for j in range(CAPA // 16)]
            m = vs[0]
            for v in vs[1:]:
                m = jnp.maximum(m, v)
            g = jnp.max(m, axis=0)
            gs = jnp.full((16,), g)
            first = jnp.int32(0)
            lane = jnp.int32(0)
            for j in range(CAPA // 16 - 1, -1, -1):
                eq = vs[j] == gs
                has = _scal(plsc.all_reduce_population_count(eq)) > 0
                lj = _scal(plsc.all_reduce_ffs(eq))
                first = jnp.where(has, jnp.int32(j), first)
                lane = jnp.where(has, lj, lane)
            vf = av_v[pl.ds(first * 16, 16)]
            av_v[pl.ds(first * 16, 16)] = jnp.where(
                iota == jnp.full((16,), lane), jnp.full((16,), NEG), vf)
            pos = first * 16 + lane
            slot = plsc.load_gather(ap_v, [jnp.full((16,), pos, jnp.int32)])
            isplat = jnp.full((16,), i, jnp.int32)
            lane0 = iota == 0
            plsc.store_scatter(selv_v, [isplat], gs, mask=lane0)
            plsc.store_scatter(sels_v, [isplat], slot, mask=lane0)
            return 0
        lax.fori_loop(0, KT, peel_body, 0)

        v0 = selv_v[pl.ds(0, 16)]
        v1 = selv_v[pl.ds(16, 16)]
        mx = jnp.max(jnp.maximum(v0, v1), axis=0)
        mxs = jnp.full((16,), mx)
        e0 = jnp.exp(v0 - mxs)
        e1 = jnp.exp(v1 - mxs)
        zs = jnp.full((16,), jnp.sum(e0, axis=0) + jnp.sum(e1, axis=0))
        selw_v[pl.ds(0, 16)] = e0 / zs
        selw_v[pl.ds(16, 16)] = e1 / zs

        pltpu.async_copy(vals_hbm.at[sels_v], vrows_v, sem2).wait()
        zf = jnp.zeros((16,), jnp.float32)
        for cb in range(DM // 16):
            acc_v[pl.ds(cb * 16, 16)] = zf

        def acc_body(i, _c):
            isplat = jnp.full((16,), i, jnp.int32)
            ws = plsc.load_gather(selw_v, [isplat])
            for cb in range(DM // 16):
                acc_v[pl.ds(cb * 16, 16)] = (
                    acc_v[pl.ds(cb * 16, 16)]
                    + ws * plsc.load_gather(vrows_v, [isplat, cb * 16 + iota]))
            return 0
        lax.fori_loop(0, KT, acc_body, 0)
        pltpu.sync_copy(acc_v, out_hbm.at[row])
        return 0

    lax.fori_loop(0, RPW, do_row, 0)


def _stage_bc(scores2, cmr, c1k, t0b, vals):
    mesh = plsc.VectorSubcoreMesh(core_axis_name="c", subcore_axis_name="s")
    kern = pl.kernel(
        _bc_body,
        out_type=jax.ShapeDtypeStruct((L, DM), jnp.float32),
        mesh=mesh,
        compiler_params=pltpu.CompilerParams(needs_layout_passes=False),
        scratch_types=[
            pltpu.VMEM((NCH,), jnp.float32),
            pltpu.VMEM((NC1,), jnp.float32),
            pltpu.VMEM((16,), jnp.float32),
            pltpu.VMEM((KT,), jnp.int32),
            pltpu.VMEM((KT,), jnp.int32),
            pltpu.VMEM((KT, 1024), jnp.float32),
            pltpu.VMEM((WCAP,), jnp.int32),
            pltpu.VMEM((CAPA,), jnp.float32),
            pltpu.VMEM((CAPA,), jnp.int32),
            pltpu.VMEM((KT,), jnp.float32),
            pltpu.VMEM((KT,), jnp.float32),
            pltpu.VMEM((KT,), jnp.int32),
            pltpu.VMEM((KT, DM), jnp.float32),
            pltpu.VMEM((DM,), jnp.float32),
            pltpu.SemaphoreType.DMA,
            pltpu.SemaphoreType.DMA,
        ],
    )
    return kern(scores2, cmr, c1k, t0b, vals)


def kernel(x, keys, vals):
    q = x.reshape(L, DM)
    qb = q.astype(jnp.bfloat16)
    kb = keys.astype(jnp.bfloat16)
    scores3, cm128, cm1k = _stage_a(qb, kb)
    cmr = jnp.transpose(cm128, (1, 0, 2)).reshape(L, NCH)
    c1k = jnp.transpose(cm1k, (1, 0, 2)).reshape(L, NC1)
    t0b = _stage_a2(c1k)
    scores2 = scores3.reshape(NC1 * L, 1024)
    out = _stage_bc(scores2, cmr, c1k, t0b, vals)
    return out.reshape(1, L, DM)


# speedup vs baseline: 1.2186x; 1.2186x over previous
import functools

import jax
import jax.numpy as jnp
from jax import lax
from jax.experimental import pallas as pl
from jax.experimental.pallas import tpu as pltpu
from jax.experimental.pallas import tpu_sc as plsc

DM = 1024
NS = 65536
KT = 32
L = 2048
ST = 1024
NT = NS // ST
NCH = NS // 128
NW = 32
RPW = L // NW
CAPC = 32
CAPA = 128
NEG = -3.0e38


def _a_body(q_ref, k_ref, s_ref, cm_ref):
    s = lax.dot_general(q_ref[...], k_ref[...], (((1,), (1,)), ((), ())),
                        preferred_element_type=jnp.float32)
    cms = []
    for j in range(ST // 128):
        blk = s[:, j * 128:(j + 1) * 128]
        s_ref[j] = blk
        cms.append(jnp.max(blk, axis=1, keepdims=True))
    cm_ref[0] = jnp.concatenate(cms, axis=1)


def _stage_a(qb, kb):
    return pl.pallas_call(
        _a_body,
        grid=(NT,),
        in_specs=[pl.BlockSpec((L, DM), lambda i: (0, 0)),
                  pl.BlockSpec((ST, DM), lambda i: (i, 0))],
        out_specs=[pl.BlockSpec((ST // 128, L, 128), lambda i: (i, 0, 0)),
                   pl.BlockSpec((1, L, ST // 128), lambda i: (i, 0, 0))],
        out_shape=[jax.ShapeDtypeStruct((NCH, L, 128), jnp.float32),
                   jax.ShapeDtypeStruct((NT, L, ST // 128), jnp.float32)],
    )(qb, kb)


def _a2_body(cm_ref, t0_ref):
    v = cm_ref[...]
    li = lax.broadcasted_iota(jnp.int32, (L, NCH), 1)
    cur = None
    for _ in range(KT):
        cur = jnp.max(v, axis=1, keepdims=True)
        sel = jnp.where(v == cur, li, NCH)
        first = jnp.min(sel, axis=1, keepdims=True)
        v = jnp.where(li == first, NEG, v)
    t0_ref[...] = jnp.broadcast_to(cur, (L, 16))


def _stage_a2(cmr):
    return pl.pallas_call(
        _a2_body,
        out_shape=jax.ShapeDtypeStruct((L, 16), jnp.float32),
    )(cmr)


def _scal(v):
    return jnp.max(v, axis=0) if v.ndim else v


def _bc_body(sc_ref, cm_hbm, t0_hbm, vals_hbm, out_hbm,
             cm_v, t0_v, cand_v, candg_v, gath_v, av_v, ap_v, selv_v, selw_v, sels_v,
             vrows_v, acc_v, sem1, sem2):
    wid = lax.axis_index("s") * 2 + lax.axis_index("c")
    iota = lax.broadcasted_iota(jnp.int32, (16,), 0)

    def do_row(r, _carry):
        row = wid * RPW + r
        pltpu.sync_copy(cm_hbm.at[row], cm_v)
        pltpu.sync_copy(t0_hbm.at[row], t0_v)
        t0 = t0_v[...]

        zz = jnp.zeros((16,), jnp.int32)
        for j in range(CAPC // 16):
            cand_v[pl.ds(j * 16, 16)] = zz

        def scan_body(j, c):
            off, tot = c
            v = cm_v[pl.ds(j * 16, 16)]
            m = v >= t0
            ids = j * 16 + iota
            idx = jnp.minimum(off + jnp.cumsum(m.astype(jnp.int32)) - 1, CAPC - 1)
            plsc.store_scatter(cand_v, [idx], ids, mask=m)
            cnt = _scal(plsc.all_reduce_population_count(m))
            return (off + cnt, tot + cnt)
        _, ntot = lax.fori_loop(0, NCH // 16, scan_body, (0, 0))
        nch = jnp.minimum(ntot, CAPC)

        for j in range(CAPC // 16):
            candg_v[pl.ds(j * 16, 16)] = cand_v[pl.ds(j * 16, 16)] * L + row

        pltpu.async_copy(sc_ref.at[candg_v], gath_v, sem1).wait()

        for j in range(CAPA // 16):
            av_v[pl.ds(j * 16, 16)] = jnp.full((16,), NEG, jnp.float32)

        def slot_body(jk, off):
            j = jk // 8
            k = jk % 8
            v = plsc.load_gather(gath_v, [jnp.full((16,), j, jnp.int32),
                                          k * 16 + iota])
            m = v >= t0
            cid = plsc.load_gather(cand_v, [jnp.full((16,), j, jnp.int32)])
            slots = cid * 128 + k * 16 + iota
            idx = off + jnp.cumsum(m.astype(jnp.int32)) - 1
            plsc.store_scatter(av_v, [idx], v, mask=m)
            plsc.store_scatter(ap_v, [idx], slots, mask=m)
            cnt = _scal(plsc.all_reduce_population_count(m))
            return jnp.minimum(off + cnt, CAPA - 16)
        lax.fori_loop(0, KT * 8, slot_body, 0)

        def peel_body(i, _c):
            vs = [av_v[pl.ds(j * 16, 16)] for j in range(CAPA // 16)]
            m = vs[0]
            for v in vs[1:]:
                m = jnp.maximum(m, v)
            g = jnp.max(m, axis=0)
            gs = jnp.full((16,), g)
            first = jnp.int32(0)
            lane = jnp.int32(0)
            for j in range(CAPA // 16 - 1, -1, -1):
                eq = vs[j] == gs
                has = _scal(plsc.all_reduce_population_count(eq)) > 0
                lj = _scal(plsc.all_reduce_ffs(eq))
                first = jnp.where(has, jnp.int32(j), first)
                lane = jnp.where(has, lj, lane)
            vf = av_v[pl.ds(first * 16, 16)]
            av_v[pl.ds(first * 16, 16)] = jnp.where(
                iota == jnp.full((16,), lane), jnp.full((16,), NEG), vf)
            pos = first * 16 + lane
            slot = plsc.load_gather(ap_v, [jnp.full((16,), pos, jnp.int32)])
            isplat = jnp.full((16,), i, jnp.int32)
            lane0 = iota == 0
            plsc.store_scatter(selv_v, [isplat], gs, mask=lane0)
            plsc.store_scatter(sels_v, [isplat], slot, mask=lane0)
            return 0
        lax.fori_loop(0, KT, peel_body, 0)

        v0 = selv_v[pl.ds(0, 16)]
        v1 = selv_v[pl.ds(16, 16)]
        mx = jnp.max(jnp.maximum(v0, v1), axis=0)
        mxs = jnp.full((16,), mx)
        e0 = jnp.exp(v0 - mxs)
        e1 = jnp.exp(v1 - mxs)
        zs = jnp.full((16,), jnp.sum(e0, axis=0) + jnp.sum(e1, axis=0))
        selw_v[pl.ds(0, 16)] = e0 / zs
        selw_v[pl.ds(16, 16)] = e1 / zs

        pltpu.async_copy(vals_hbm.at[sels_v], vrows_v, sem2).wait()
        zf = jnp.zeros((16,), jnp.float32)
        for cb in range(DM // 16):
            acc_v[pl.ds(cb * 16, 16)] = zf

        def acc_body(i, _c):
            isplat = jnp.full((16,), i, jnp.int32)
            ws = plsc.load_gather(selw_v, [isplat])
            for cb in range(DM // 16):
                acc_v[pl.ds(cb * 16, 16)] = (
                    acc_v[pl.ds(cb * 16, 16)]
                    + ws * plsc.load_gather(vrows_v, [isplat, cb * 16 + iota]))
            return 0
        lax.fori_loop(0, KT, acc_body, 0)
        pltpu.sync_copy(acc_v, out_hbm.at[row])
        return 0

    lax.fori_loop(0, RPW, do_row, 0)


def _stage_bc(scores2, cmr, t0b, vals):
    mesh = plsc.VectorSubcoreMesh(core_axis_name="c", subcore_axis_name="s")
    kern = pl.kernel(
        _bc_body,
        out_type=jax.ShapeDtypeStruct((L, DM), jnp.float32),
        mesh=mesh,
        compiler_params=pltpu.CompilerParams(needs_layout_passes=False),
        scratch_types=[
            pltpu.VMEM((NCH,), jnp.float32),
            pltpu.VMEM((16,), jnp.float32),
            pltpu.VMEM((CAPC,), jnp.int32),
            pltpu.VMEM((CAPC,), jnp.int32),
            pltpu.VMEM((CAPC, 128), jnp.float32),
            pltpu.VMEM((CAPA,), jnp.float32),
            pltpu.VMEM((CAPA,), jnp.int32),
            pltpu.VMEM((KT,), jnp.float32),
            pltpu.VMEM((KT,), jnp.float32),
            pltpu.VMEM((KT,), jnp.int32),
            pltpu.VMEM((KT, DM), jnp.float32),
            pltpu.VMEM((DM,), jnp.float32),
            pltpu.SemaphoreType.DMA,
            pltpu.SemaphoreType.DMA,
        ],
    )
    return kern(scores2, cmr, t0b, vals)


def kernel(x, keys, vals):
    q = x.reshape(L, DM)
    qb = q.astype(jnp.bfloat16)
    kb = keys.astype(jnp.bfloat16)
    scores3, cm128 = _stage_a(qb, kb)
    cmr = jnp.transpose(cm128, (1, 0, 2)).reshape(L, NCH)
    t0b = _stage_a2(cmr)
    scores2 = scores3.reshape(NCH * L, 128)
    out = _stage_bc(scores2, cmr, t0b, vals)
    return out.reshape(1, L, DM)


# speedup vs baseline: 1.2506x; 1.0263x over previous
import functools

import jax
import jax.numpy as jnp
from jax import lax
from jax.experimental import pallas as pl
from jax.experimental.pallas import tpu as pltpu
from jax.experimental.pallas import tpu_sc as plsc

DM = 1024
NS = 65536
KT = 32
L = 2048
ST = 1024
NT = NS // ST
NCH = NS // 128
NW = 32
RPW = L // NW
CAPC = 32
CAPA = 128
NEG = -3.0e38


def _a_body(q_ref, k_ref, s_ref, cm_ref):
    s = lax.dot_general(q_ref[...], k_ref[...], (((1,), (1,)), ((), ())),
                        preferred_element_type=jnp.float32)
    cms = []
    for j in range(ST // 128):
        blk = s[:, j * 128:(j + 1) * 128]
        s_ref[j] = blk
        cms.append(jnp.max(blk, axis=1, keepdims=True))
    cm_ref[0] = jnp.concatenate(cms, axis=1)


def _stage_a(qb, kb):
    return pl.pallas_call(
        _a_body,
        grid=(NT,),
        in_specs=[pl.BlockSpec((L, DM), lambda i: (0, 0)),
                  pl.BlockSpec((ST, DM), lambda i: (i, 0))],
        out_specs=[pl.BlockSpec((ST // 128, L, 128), lambda i: (i, 0, 0)),
                   pl.BlockSpec((1, L, ST // 128), lambda i: (i, 0, 0))],
        out_shape=[jax.ShapeDtypeStruct((NCH, L, 128), jnp.float32),
                   jax.ShapeDtypeStruct((NT, L, ST // 128), jnp.float32)],
    )(qb, kb)


def _a2_body(cm_ref, t0_ref):
    v = cm_ref[...]
    li = lax.broadcasted_iota(jnp.int32, (L, NCH), 1)
    cur = None
    for _ in range(KT):
        cur = jnp.max(v, axis=1, keepdims=True)
        sel = jnp.where(v == cur, li, NCH)
        first = jnp.min(sel, axis=1, keepdims=True)
        v = jnp.where(li == first, NEG, v)
    t0_ref[...] = jnp.broadcast_to(cur, (L, 16))


def _stage_a2(cmr):
    return pl.pallas_call(
        _a2_body,
        out_shape=jax.ShapeDtypeStruct((L, 16), jnp.float32),
    )(cmr)


def _scal(v):
    return jnp.max(v, axis=0) if v.ndim else v


def _bc_body(sc_ref, cm_hbm, t0_hbm, vals_hbm, out_hbm,
             cm_v, t0s_v, t0_v, cand_v, candg_v, gath_v, av_v, ap_v,
             selv_v, selw_v, sels_v, vrows_v, acc_v, sem1, sem2):
    wid = lax.axis_index("s") * 2 + lax.axis_index("c")
    iota = lax.broadcasted_iota(jnp.int32, (16,), 0)

    def front(rr, par):
        row = wid * RPW + rr
        base = par * CAPC
        pltpu.sync_copy(cm_hbm.at[row], cm_v)
        pltpu.sync_copy(t0_hbm.at[row], t0s_v)
        t0 = t0s_v[...]
        t0_v[pl.ds(par * 16, 16)] = t0

        zz = jnp.zeros((16,), jnp.int32)
        for j in range(CAPC // 16):
            cand_v[pl.ds(base + j * 16, 16)] = zz

        def scan_body(j, off):
            v = cm_v[pl.ds(j * 16, 16)]
            m = v >= t0
            ids = j * 16 + iota
            idx = base + jnp.minimum(
                off + jnp.cumsum(m.astype(jnp.int32)) - 1, CAPC - 1)
            plsc.store_scatter(cand_v, [idx], ids, mask=m)
            cnt = _scal(plsc.all_reduce_population_count(m))
            return off + cnt
        lax.fori_loop(0, NCH // 16, scan_body, 0)

        for j in range(CAPC // 16):
            candg_v[pl.ds(base + j * 16, 16)] = (
                cand_v[pl.ds(base + j * 16, 16)] * L + row)
        pltpu.make_async_copy(
            sc_ref.at[candg_v.at[pl.ds(base, CAPC)]],
            gath_v.at[pl.ds(base, CAPC)], sem1).start()

    front(0, 0)

    def do_row(r, _carry):
        row = wid * RPW + r
        par = jnp.bitwise_and(r, 1)
        base = par * CAPC
        t0 = t0_v[pl.ds(par * 16, 16)]

        pltpu.make_async_copy(
            sc_ref.at[candg_v.at[pl.ds(base, CAPC)]],
            gath_v.at[pl.ds(base, CAPC)], sem1).wait()

        @pl.when(r < RPW - 1)
        def _():
            front(r + 1, 1 - par)

        for j in range(CAPA // 16):
            av_v[pl.ds(j * 16, 16)] = jnp.full((16,), NEG, jnp.float32)

        def slot_body(jk, off):
            j = jk // 8
            k = jk % 8
            v = plsc.load_gather(gath_v,
                                 [jnp.full((16,), base + j, jnp.int32),
                                  k * 16 + iota])
            m = v >= t0
            cid = plsc.load_gather(cand_v,
                                   [jnp.full((16,), base + j, jnp.int32)])
            slots = cid * 128 + k * 16 + iota
            idx = off + jnp.cumsum(m.astype(jnp.int32)) - 1
            plsc.store_scatter(av_v, [idx], v, mask=m)
            plsc.store_scatter(ap_v, [idx], slots, mask=m)
            cnt = _scal(plsc.all_reduce_population_count(m))
            return jnp.minimum(off + cnt, CAPA - 16)
        lax.fori_loop(0, KT * 8, slot_body, 0)

        def peel_body(i, _c):
            vs = [av_v[pl.ds(j * 16, 16)] for j in range(CAPA // 16)]
            m = vs[0]
            for v in vs[1:]:
                m = jnp.maximum(m, v)
            g = jnp.max(m, axis=0)
            gs = jnp.full((16,), g)
            first = jnp.int32(0)
            lane = jnp.int32(0)
            for j in range(CAPA // 16 - 1, -1, -1):
                eq = vs[j] == gs
                has = _scal(plsc.all_reduce_population_count(eq)) > 0
                lj = _scal(plsc.all_reduce_ffs(eq))
                first = jnp.where(has, jnp.int32(j), first)
                lane = jnp.where(has, lj, lane)
            vf = av_v[pl.ds(first * 16, 16)]
            av_v[pl.ds(first * 16, 16)] = jnp.where(
                iota == jnp.full((16,), lane), jnp.full((16,), NEG), vf)
            pos = first * 16 + lane
            slot = plsc.load_gather(ap_v, [jnp.full((16,), pos, jnp.int32)])
            isplat = jnp.full((16,), i, jnp.int32)
            lane0 = iota == 0
            plsc.store_scatter(selv_v, [isplat], gs, mask=lane0)
            plsc.store_scatter(sels_v, [isplat], slot, mask=lane0)
            return 0
        lax.fori_loop(0, KT, peel_body, 0)

        v0 = selv_v[pl.ds(0, 16)]
        v1 = selv_v[pl.ds(16, 16)]
        mx = jnp.max(jnp.maximum(v0, v1), axis=0)
        mxs = jnp.full((16,), mx)
        e0 = jnp.exp(v0 - mxs)
        e1 = jnp.exp(v1 - mxs)
        zs = jnp.full((16,), jnp.sum(e0, axis=0) + jnp.sum(e1, axis=0))
        selw_v[pl.ds(0, 16)] = e0 / zs
        selw_v[pl.ds(16, 16)] = e1 / zs

        pltpu.async_copy(vals_hbm.at[sels_v], vrows_v, sem2).wait()
        zf = jnp.zeros((16,), jnp.float32)
        for cb in range(DM // 16):
            acc_v[pl.ds(cb * 16, 16)] = zf

        def acc_body(i, _c):
            isplat = jnp.full((16,), i, jnp.int32)
            ws = plsc.load_gather(selw_v, [isplat])
            for cb in range(DM // 16):
                acc_v[pl.ds(cb * 16, 16)] = (
                    acc_v[pl.ds(cb * 16, 16)]
                    + ws * plsc.load_gather(vrows_v, [isplat, cb * 16 + iota]))
            return 0
        lax.fori_loop(0, KT, acc_body, 0)
        pltpu.sync_copy(acc_v, out_hbm.at[row])
        return 0

    lax.fori_loop(0, RPW, do_row, 0)


def _stage_bc(scores2, cmr, t0b, vals):
    mesh = plsc.VectorSubcoreMesh(core_axis_name="c", subcore_axis_name="s")
    kern = pl.kernel(
        _bc_body,
        out_type=jax.ShapeDtypeStruct((L, DM), jnp.float32),
        mesh=mesh,
        compiler_params=pltpu.CompilerParams(needs_layout_passes=False),
        scratch_types=[
            pltpu.VMEM((NCH,), jnp.float32),
            pltpu.VMEM((16,), jnp.float32),
            pltpu.VMEM((32,), jnp.float32),
            pltpu.VMEM((2 * CAPC,), jnp.int32),
            pltpu.VMEM((2 * CAPC,), jnp.int32),
            pltpu.VMEM((2 * CAPC, 128), jnp.float32),
            pltpu.VMEM((CAPA,), jnp.float32),
            pltpu.VMEM((CAPA,), jnp.int32),
            pltpu.VMEM((KT,), jnp.float32),
            pltpu.VMEM((KT,), jnp.float32),
            pltpu.VMEM((KT,), jnp.int32),
            pltpu.VMEM((KT, DM), jnp.float32),
            pltpu.VMEM((DM,), jnp.float32),
            pltpu.SemaphoreType.DMA,
            pltpu.SemaphoreType.DMA,
        ],
    )
    return kern(scores2, cmr, t0b, vals)


def kernel(x, keys, vals):
    q = x.reshape(L, DM)
    qb = q.astype(jnp.bfloat16)
    kb = keys.astype(jnp.bfloat16)
    scores3, cm128 = _stage_a(qb, kb)
    cmr = jnp.transpose(cm128, (1, 0, 2)).reshape(L, NCH)
    t0b = _stage_a2(cmr)
    scores2 = scores3.reshape(NCH * L, 128)
    out = _stage_bc(scores2, cmr, t0b, vals)
    return out.reshape(1, L, DM)
